# baseline (speedup 1.0000x reference)
import functools

import jax
import jax.numpy as jnp
from jax import lax
from jax.experimental import pallas as pl
from jax.experimental.pallas import tpu as pltpu
from jax.experimental.pallas import tpu_sc as plsc

F = 2
B = 16384
H = 20
D = 32
V = 1000000
V4 = V // 4
N = B * H
NC = 2
NS = 16
NW = NC * NS
PER_W = N // NW
CH = 1024
NCH = PER_W // CH
NBUF = 2

K1CH = 64
K1ROWS = K1CH * 4
K1CHUNKS = -(-V4 // K1CH)
K1W = 16
K1PAIRS = 123
K1IT = 2 * K1PAIRS

_mesh = plsc.VectorSubcoreMesh(core_axis_name="c", subcore_axis_name="s")


@functools.partial(
    pl.kernel,
    mesh=_mesh,
    out_type=[jax.ShapeDtypeStruct((V4, 128), jnp.float32),
              jax.ShapeDtypeStruct((V4, 128), jnp.float32)],
    scratch_types=[
        pltpu.VMEM((2, K1ROWS, D), jnp.float32),
        pltpu.VMEM((2, K1CH, 128), jnp.float32),
        pltpu.SemaphoreType.DMA,
        pltpu.SemaphoreType.DMA,
        pltpu.SemaphoreType.DMA,
        pltpu.SemaphoreType.DMA,
    ],
)
def _repack(t0_hbm, t1_hbm, p0_hbm, p1_hbm, in_v, out_v,
            isem_a, isem_b, osem_a, osem_b):
    wid = lax.axis_index("s") * NC + lax.axis_index("c")
    isems = (isem_a, isem_b)
    osems = (osem_a, osem_b)

    def run(tab, pk, w):
        c00 = w * K1IT
        nv = jnp.minimum(K1IT, K1CHUNKS - c00)

        def line0(it):
            return jnp.minimum((c00 + it) * K1CH, V4 - K1CH)

        def fire_in(slot, it):
            @pl.when(it < nv)
            def _():
                pltpu.async_copy(tab.at[pl.ds(line0(it) * 4, K1ROWS), :],
                                 in_v.at[slot], isems[slot])

        def proc(slot, it):
            @pl.when(it < nv)
            def _():
                pltpu.make_async_copy(tab.at[pl.ds(0, K1ROWS), :],
                                      in_v.at[slot], isems[slot]).wait()

                @pl.when(it >= 2)
                def _():
                    pltpu.make_async_copy(pk.at[pl.ds(0, K1CH)],
                                          out_v.at[slot],
                                          osems[slot]).wait()

                def slab(s, carry):
                    for q in range(4):
                        for h in range(2):
                            out_v[slot, s, pl.ds(q * 32 + h * 16, 16)] = (
                                in_v[slot, 4 * s + q, pl.ds(h * 16, 16)])
                    return carry

                lax.fori_loop(0, K1CH, slab, 0)
                pltpu.async_copy(out_v.at[slot],
                                 pk.at[pl.ds(line0(it), K1CH)], osems[slot])

        fire_in(0, 0)

        def pair(jp, carry):
            it1 = 2 * jp + 1
            fire_in(1, it1)
            proc(0, it1 - 1)
            fire_in(0, it1 + 1)
            proc(1, it1)
            return carry

        lax.fori_loop(0, K1PAIRS, pair, 0)

        for slot in range(2):
            @pl.when(nv >= slot + 1)
            def _():
                pltpu.make_async_copy(pk.at[pl.ds(0, K1CH)],
                                      out_v.at[slot], osems[slot]).wait()

    @pl.when(wid < K1W)
    def _():
        run(t0_hbm, p0_hbm, wid)

    @pl.when(wid >= K1W)
    def _():
        run(t1_hbm, p1_hbm, wid - K1W)


@functools.partial(
    pl.kernel,
    mesh=_mesh,
    compiler_params=pltpu.CompilerParams(use_tc_tiling_on_sc=False),
    out_type=jax.ShapeDtypeStruct((F * N, D), jnp.float32),
    scratch_types=[
        pltpu.VMEM((NBUF, CH), jnp.int32),
        pltpu.VMEM((NBUF, CH, D), jnp.float32),
        pltpu.SemaphoreType.DMA,
        pltpu.SemaphoreType.DMA,
    ],
)
def _lookup(idx_hbm, t0_hbm, t1_hbm, out_hbm, idx_v, rows_v, sem0, sem1):
    wid = lax.axis_index("s") * NC + lax.axis_index("c")
    base = wid * PER_W
    tables = (t0_hbm, t1_hbm)
    sems = (sem0, sem1)
    chunks = [(f, j) for f in range(F) for j in range(NCH)]

    def start(slot, f, j):
        off = f * N + base + j * CH
        pltpu.sync_copy(idx_hbm.at[pl.ds(off, CH)], idx_v.at[slot])
        return pltpu.async_copy(
            tables[f].at[idx_v.at[slot]], rows_v.at[slot], sems[slot])

    inflight = {0: start(0, *chunks[0])}
    for i, (f, j) in enumerate(chunks):
        slot = i % NBUF
        if i + 1 < len(chunks):
            nslot = (i + 1) % NBUF
            inflight[nslot] = start(nslot, *chunks[i + 1])
        inflight[slot].wait()
        off = f * N + base + j * CH
        pltpu.sync_copy(rows_v.at[slot], out_hbm.at[pl.ds(off, CH)])


def kernel(indices, table_0, table_1):
    idx = indices.reshape(F * N).astype(jnp.int32)
    p0, p1 = _repack(table_0, table_1)
    out = _lookup(idx, p0.reshape(V, D), p1.reshape(V, D))
    return out.reshape(F, B, H, D)


# speedup vs baseline: 1.1588x; 1.1588x over previous
import functools

import jax
import jax.numpy as jnp
from jax import lax
from jax.experimental import pallas as pl
from jax.experimental.pallas import tpu as pltpu
from jax.experimental.pallas import tpu_sc as plsc

F = 2
B = 16384
H = 20
D = 32
V = 1000000
N = B * H
NC = 2
NS = 16
NW = NC * NS
PER_W = N // NW
CH = 1024
NCH = PER_W // CH
NBUF = 2

_mesh = plsc.VectorSubcoreMesh(core_axis_name="c", subcore_axis_name="s")


@functools.partial(
    pl.kernel,
    mesh=_mesh,
    compiler_params=pltpu.CompilerParams(use_tc_tiling_on_sc=False),
    out_type=jax.ShapeDtypeStruct((F, N, D), jnp.float32),
    scratch_types=[
        pltpu.VMEM((NBUF, CH), jnp.int32),
        pltpu.VMEM((NBUF, CH, D), jnp.float32),
        pltpu.SemaphoreType.DMA,
        pltpu.SemaphoreType.DMA,
    ],
)
def _lookup(idx_hbm, t0_hbm, t1_hbm, out_hbm, idx_v, rows_v, sem0, sem1):
    wid = lax.axis_index("s") * NC + lax.axis_index("c")
    base = wid * PER_W
    tables = (t0_hbm, t1_hbm)
    sems = (sem0, sem1)
    chunks = [(f, j) for f in range(F) for j in range(NCH)]

    def start(slot, f, j):
        off = f * N + base + j * CH
        pltpu.sync_copy(idx_hbm.at[pl.ds(off, CH)], idx_v.at[slot])
        return pltpu.async_copy(
            tables[f].at[idx_v.at[slot]], rows_v.at[slot], sems[slot])

    inflight = {0: start(0, *chunks[0])}
    for i, (f, j) in enumerate(chunks):
        slot = i % NBUF
        if i + 1 < len(chunks):
            nslot = (i + 1) % NBUF
            inflight[nslot] = start(nslot, *chunks[i + 1])
        inflight[slot].wait()
        off = base + j * CH
        pltpu.sync_copy(rows_v.at[slot], out_hbm.at[f, pl.ds(off, CH)])


def kernel(indices, table_0, table_1):
    idx = indices.reshape(F * N).astype(jnp.int32)
    out = _lookup(idx, table_0, table_1)
    return out.reshape(F, B, H, D)


# speedup vs baseline: 1.1613x; 1.0021x over previous
import functools

import jax
import jax.numpy as jnp
from jax import lax
from jax.experimental import pallas as pl
from jax.experimental.pallas import tpu as pltpu
from jax.experimental.pallas import tpu_sc as plsc

F = 2
B = 16384
H = 20
D = 32
V = 1000000
N = B * H
NC = 2
NS = 16
NW = NC * NS
PER_W = N // NW
CH = 1024
NCH = PER_W // CH
NBUF = 2

_mesh = plsc.VectorSubcoreMesh(core_axis_name="c", subcore_axis_name="s")


@functools.partial(
    pl.kernel,
    mesh=_mesh,
    compiler_params=pltpu.CompilerParams(use_tc_tiling_on_sc=False),
    out_type=jax.ShapeDtypeStruct((N, D), jnp.float32),
    scratch_types=[
        pltpu.VMEM((NBUF, CH), jnp.int32),
        pltpu.VMEM((NBUF, CH, D), jnp.float32),
        pltpu.SemaphoreType.DMA,
        pltpu.SemaphoreType.DMA,
    ],
)
def _lookup1(idx_hbm, tab_hbm, out_hbm, idx_v, rows_v, sem0, sem1):
    wid = lax.axis_index("s") * NC + lax.axis_index("c")
    base = wid * PER_W
    sems = (sem0, sem1)

    def start(slot, j):
        off = base + j * CH
        pltpu.sync_copy(idx_hbm.at[pl.ds(off, CH)], idx_v.at[slot])
        return pltpu.async_copy(
            tab_hbm.at[idx_v.at[slot]], rows_v.at[slot], sems[slot])

    inflight = {0: start(0, 0)}
    for j in range(NCH):
        slot = j % NBUF
        if j + 1 < NCH:
            inflight[(j + 1) % NBUF] = start((j + 1) % NBUF, j + 1)
        inflight[slot].wait()
        pltpu.sync_copy(rows_v.at[slot], out_hbm.at[pl.ds(base + j * CH, CH)])


def kernel(indices, table_0, table_1):
    idx = indices.reshape(F, N).astype(jnp.int32)
    o0 = _lookup1(idx[0], table_0)
    o1 = _lookup1(idx[1], table_1)
    return jnp.stack([o0.reshape(B, H, D), o1.reshape(B, H, D)], axis=0)
